# baseline (speedup 1.0000x reference)
import functools

import jax
import jax.numpy as jnp
from jax import lax
from jax.experimental import pallas as pl
from jax.experimental.pallas import tpu as pltpu
from jax.experimental.pallas import tpu_sc as plsc

D = 64
DP = 128
B = 4096
CTX = 20
SAMP = 20
NC, NSUB = 2, 16
NW = NC * NSUB
BPW = B // NW
NB = 8
NBLK = BPW // NB
ROWS = NB * CTX
VL = 16
OUTW = 2 * VL


def _dot_partial(acc, sr, r, hb):
    p = acc[0] * sr[r, pl.ds(hb, VL)]
    for k in range(1, 4):
        p = p + acc[k] * sr[r, pl.ds(hb + k * VL, VL)]
    return p


def _half(idx_ref, off):
    v = idx_ref[pl.ds(off, VL)]
    return (v[0] & 1) << 6


def _sc_body(doc_hbm, cidx_hbm, sidx_hbm, para_hbm, word_hbm, outt_hbm,
             out_hbm, didx, cidx, sidx, didx2, cidx2, sidx2,
             drows, crows, srows, lg, pbuf,
             gsem0, gsem1, osem0, osem1):
    wid = lax.axis_index("s") * NC + lax.axis_index("c")
    wbase = wid * BPW

    pltpu.sync_copy(doc_hbm.at[pl.ds(wbase, BPW)], didx.at[pl.ds(0, BPW)])
    pltpu.sync_copy(cidx_hbm.at[pl.ds(wbase * CTX, BPW * CTX)],
                    cidx.at[pl.ds(0, BPW * CTX)])
    pltpu.sync_copy(sidx_hbm.at[pl.ds(wbase * SAMP, BPW * SAMP)],
                    sidx.at[pl.ds(0, BPW * SAMP)])

    for t in range(BPW // VL):
        didx2[pl.ds(t * VL, VL)] = lax.shift_right_logical(
            didx[pl.ds(t * VL, VL)], 1)

    def shift_body(t, carry):
        o = t * VL
        cidx2[pl.ds(o, VL)] = lax.shift_right_logical(cidx[pl.ds(o, VL)], 1)
        sidx2[pl.ds(o, VL)] = lax.shift_right_logical(sidx[pl.ds(o, VL)], 1)
        return carry

    lax.fori_loop(0, BPW * CTX // VL, shift_body, 0)

    gsems = (gsem0, gsem1)
    osems = (osem0, osem1)

    def fire(m, pslot):
        hs = [pltpu.make_async_copy(
            para_hbm.at[didx2.at[pl.ds(pl.multiple_of(m * NB, 8), NB)]],
            drows.at[pslot], gsems[pslot])]
        for off, sz in ((0, 128), (128, 32)):
            hs.append(pltpu.make_async_copy(
                word_hbm.at[cidx2.at[pl.ds(
                    pl.multiple_of(m * ROWS + off, 8), sz)]],
                crows.at[pslot].at[pl.ds(off, sz)], gsems[pslot]))
            hs.append(pltpu.make_async_copy(
                outt_hbm.at[sidx2.at[pl.ds(
                    pl.multiple_of(m * ROWS + off, 8), sz)]],
                srows.at[pslot].at[pl.ds(off, sz)], gsems[pslot]))
        return hs

    def compute(m, pslot):
        dr = drows.at[pslot]
        cr = crows.at[pslot]
        sr = srows.at[pslot]
        lgs = lg.at[pslot]
        lane16 = lax.iota(jnp.int32, VL) * VL
        dbase = m * NB
        rbase = m * ROWS

        def body(b, carry):
            hb = _half(didx, dbase + b)
            acc = [dr[b, pl.ds(hb + k * VL, VL)] for k in range(4)]
            for c in range(CTX):
                r = b * CTX + c
                hb = _half(cidx, rbase + r)
                for k in range(4):
                    acc[k] = acc[k] + cr[r, pl.ds(hb + k * VL, VL)]
            for s in range(SAMP):
                r = b * SAMP + s
                pbuf[pl.ds(s * VL, VL)] = _dot_partial(
                    acc, sr, r, _half(sidx, rbase + r))
            for g in range(2):
                v = plsc.load_gather(pbuf, [lane16 + g * VL * VL])
                for k in range(1, VL):
                    v = v + plsc.load_gather(pbuf, [lane16 + (g * VL * VL + k)])
                lgs[b, pl.ds(g * VL, VL)] = v
            return carry

        lax.fori_loop(0, NB, body, 0)

    def write_out(m, pslot):
        return pltpu.make_async_copy(
            lg.at[pslot],
            out_hbm.at[pl.ds(pl.multiple_of(wbase + m * NB, 8), NB)],
            osems[pslot])

    for h in fire(0, 0):
        h.start()
    for h in fire(1, 1):
        h.start()

    def block_pair(i, carry):
        for pslot in range(2):
            m = 2 * i + pslot
            for h in fire(m, pslot):
                h.wait()

            @pl.when(i > 0)
            def _():
                write_out(m, pslot).wait()

            compute(m, pslot)
            write_out(m, pslot).start()

            @pl.when(m + 2 < NBLK)
            def _():
                for h in fire(m + 2, pslot):
                    h.start()
        return carry

    lax.fori_loop(0, NBLK // 2, block_pair, 0)
    write_out(NBLK - 2, 0).wait()
    write_out(NBLK - 1, 1).wait()


_sc_kernel = functools.partial(
    pl.kernel,
    out_type=jax.ShapeDtypeStruct((B, OUTW), jnp.float32),
    mesh=plsc.VectorSubcoreMesh(core_axis_name="c", subcore_axis_name="s"),
    compiler_params=pltpu.CompilerParams(needs_layout_passes=False),
    scratch_types=[
        pltpu.VMEM((BPW + VL,), jnp.int32),
        pltpu.VMEM((BPW * CTX + VL,), jnp.int32),
        pltpu.VMEM((BPW * SAMP + VL,), jnp.int32),
        pltpu.VMEM((BPW,), jnp.int32),
        pltpu.VMEM((BPW * CTX,), jnp.int32),
        pltpu.VMEM((BPW * SAMP,), jnp.int32),
        pltpu.VMEM((2, NB, DP), jnp.float32),
        pltpu.VMEM((2, ROWS, DP), jnp.float32),
        pltpu.VMEM((2, ROWS, DP), jnp.float32),
        pltpu.VMEM((2, NB, OUTW), jnp.float32),
        pltpu.VMEM((2 * VL * VL,), jnp.float32),
        pltpu.SemaphoreType.DMA,
        pltpu.SemaphoreType.DMA,
        pltpu.SemaphoreType.DMA,
        pltpu.SemaphoreType.DMA,
    ],
)(_sc_body)


_N = 100000
_CH = 2048


def _tr_body(a_ref, b_ref, c_ref, ta_ref, tb_ref, tc_ref):
    ta_ref[...] = a_ref[...].T
    tb_ref[...] = b_ref[...].T
    tc_ref[...] = c_ref[...].T


_transpose3 = pl.pallas_call(
    _tr_body,
    grid=((_N + _CH - 1) // _CH,),
    in_specs=[pl.BlockSpec((D, _CH), lambda i: (0, i)) for _ in range(3)],
    out_specs=[pl.BlockSpec((_CH, D), lambda i: (i, 0)) for _ in range(3)],
    out_shape=[jax.ShapeDtypeStruct((_N, D), jnp.float32) for _ in range(3)],
)


def kernel(doc_ids, context_ids, sample_ids, paragraph_matrix, word_matrix,
           outputs):
    doc_i = doc_ids.astype(jnp.int32)
    ctx_i = context_ids.astype(jnp.int32).reshape(-1)
    samp_i = sample_ids.astype(jnp.int32).reshape(-1)
    tp, tw, to = _transpose3(paragraph_matrix.T, word_matrix.T, outputs)
    para2 = tp.reshape(-1, DP)
    word2 = tw.reshape(-1, DP)
    outt2 = to.reshape(-1, DP)
    padded = _sc_kernel(doc_i, ctx_i, samp_i, para2, word2, outt2)
    return padded[:, :SAMP]


# speedup vs baseline: 1.1246x; 1.1246x over previous
import functools

import jax
import jax.numpy as jnp
from jax import lax
from jax.experimental import pallas as pl
from jax.experimental.pallas import tpu as pltpu
from jax.experimental.pallas import tpu_sc as plsc

D = 64
B = 4096
CTX = 20
SAMP = 20
NC, NSUB = 2, 16
NW = NC * NSUB
BPW = B // NW
NB = 16
NBLK = BPW // NB
ROWS = NB * CTX
VL = 16


def _dot_partial(acc, sr, r):
    p = acc[0] * sr[r, pl.ds(0, VL)]
    for k in range(1, 4):
        p = p + acc[k] * sr[r, pl.ds(k * VL, VL)]
    return p


def _sc_body(doc_hbm, cidx_hbm, sidx_hbm, para_hbm, word_hbm, outt_hbm,
             out_hbm, didx, cidx, sidx, drows, crows, srows, lg, pbuf,
             gsem0, gsem1, osem0, osem1):
    wid = lax.axis_index("s") * NC + lax.axis_index("c")
    wbase = wid * BPW

    pltpu.sync_copy(doc_hbm.at[pl.ds(wbase, BPW)], didx)
    pltpu.sync_copy(cidx_hbm.at[pl.ds(wbase * CTX, BPW * CTX)], cidx)
    pltpu.sync_copy(sidx_hbm.at[pl.ds(wbase * SAMP, BPW * SAMP)], sidx)

    gsems = (gsem0, gsem1)
    osems = (osem0, osem1)

    def fire(j, slot):
        base = j * NB
        hs = [pltpu.async_copy(
            para_hbm.at[didx.at[pl.ds(base, NB)]], drows.at[slot],
            gsems[slot])]
        for off, sz in ((0, 128), (128, 128), (256, 64)):
            hs.append(pltpu.async_copy(
                word_hbm.at[cidx.at[pl.ds(base * CTX + off, sz)]],
                crows.at[slot].at[pl.ds(off, sz)], gsems[slot]))
            hs.append(pltpu.async_copy(
                outt_hbm.at[sidx.at[pl.ds(base * SAMP + off, sz)]],
                srows.at[slot].at[pl.ds(off, sz)], gsems[slot]))
        return hs

    def compute(slot):
        dr = drows.at[slot]
        cr = crows.at[slot]
        sr = srows.at[slot]
        lgs = lg.at[slot]
        lane16 = lax.iota(jnp.int32, VL) * VL

        def body(b, carry):
            acc = [dr[b, pl.ds(k * VL, VL)] for k in range(4)]
            for c in range(CTX):
                r = b * CTX + c
                for k in range(4):
                    acc[k] = acc[k] + cr[r, pl.ds(k * VL, VL)]
            for s in range(SAMP):
                pbuf[pl.ds(s * VL, VL)] = _dot_partial(acc, sr, b * SAMP + s)
            for g in range(2):
                r = plsc.load_gather(pbuf, [lane16 + g * VL * VL])
                for k in range(1, VL):
                    r = r + plsc.load_gather(pbuf, [lane16 + (g * VL * VL + k)])
                lgs[b, pl.ds(g * VL, VL)] = r
            return carry

        lax.fori_loop(0, NB, body, 0)

    handles = [None, None]
    out_handles = [None, None]
    handles[0] = fire(0, 0)
    for j in range(NBLK):
        slot = j & 1
        if j + 1 < NBLK:
            handles[1 - slot] = fire(j + 1, 1 - slot)
        for h in handles[slot]:
            h.wait()
        if out_handles[slot] is not None:
            out_handles[slot].wait()
        compute(slot)
        out_handles[slot] = pltpu.async_copy(
            lg.at[slot], out_hbm.at[pl.ds(wbase + j * NB, NB)], osems[slot])
    for oh in out_handles:
        if oh is not None:
            oh.wait()


_sc_kernel = functools.partial(
    pl.kernel,
    out_type=jax.ShapeDtypeStruct((B, 2 * VL), jnp.float32),
    mesh=plsc.VectorSubcoreMesh(core_axis_name="c", subcore_axis_name="s"),
    compiler_params=pltpu.CompilerParams(
        needs_layout_passes=False, use_tc_tiling_on_sc=False),
    scratch_types=[
        pltpu.VMEM((BPW,), jnp.int32),
        pltpu.VMEM((BPW * CTX,), jnp.int32),
        pltpu.VMEM((BPW * SAMP,), jnp.int32),
        pltpu.VMEM((2, NB, D), jnp.float32),
        pltpu.VMEM((2, ROWS, D), jnp.float32),
        pltpu.VMEM((2, ROWS, D), jnp.float32),
        pltpu.VMEM((2, NB, 2 * VL), jnp.float32),
        pltpu.VMEM((2 * VL * VL,), jnp.float32),
        pltpu.SemaphoreType.DMA,
        pltpu.SemaphoreType.DMA,
        pltpu.SemaphoreType.DMA,
        pltpu.SemaphoreType.DMA,
    ],
)(_sc_body)


_N = 100000
_CH = 2048


def _tr_body(a_ref, b_ref, c_ref, ta_ref, tb_ref, tc_ref):
    ta_ref[...] = a_ref[...].T
    tb_ref[...] = b_ref[...].T
    tc_ref[...] = c_ref[...].T


_transpose3 = pl.pallas_call(
    _tr_body,
    grid=((_N + _CH - 1) // _CH,),
    in_specs=[pl.BlockSpec((D, _CH), lambda i: (0, i)) for _ in range(3)],
    out_specs=[pl.BlockSpec((_CH, D), lambda i: (i, 0)) for _ in range(3)],
    out_shape=[jax.ShapeDtypeStruct((_N, D), jnp.float32) for _ in range(3)],
)


def kernel(doc_ids, context_ids, sample_ids, paragraph_matrix, word_matrix,
           outputs):
    doc_i = doc_ids.astype(jnp.int32)
    ctx_i = context_ids.astype(jnp.int32).reshape(-1)
    samp_i = sample_ids.astype(jnp.int32).reshape(-1)
    tp, tw, to = _transpose3(paragraph_matrix.T, word_matrix.T, outputs)
    padded = _sc_kernel(doc_i, ctx_i, samp_i, tp, tw, to)
    return padded[:, :SAMP]


# speedup vs baseline: 1.3512x; 1.2015x over previous
import functools

import jax
import jax.numpy as jnp
from jax import lax
from jax.experimental import pallas as pl
from jax.experimental.pallas import tpu as pltpu
from jax.experimental.pallas import tpu_sc as plsc

D = 64
B = 4096
CTX = 20
SAMP = 20
NC, NSUB = 2, 16
NW = NC * NSUB
BPW = B // NW
NB = 16
NBLK = BPW // NB
ROWS = NB * CTX
VL = 16


def _dot_partial(acc, sr, r):
    p = acc[0] * sr[r, pl.ds(0, VL)]
    for k in range(1, 4):
        p = p + acc[k] * sr[r, pl.ds(k * VL, VL)]
    return p


def _sc_body(doc_hbm, cidx_hbm, sidx_hbm, para_hbm, word_hbm, outt_hbm,
             out_hbm, didx, cidx, sidx, drows, crows, srows, lg, pbuf,
             gsem0, gsem1, osem0, osem1):
    wid = lax.axis_index("s") * NC + lax.axis_index("c")
    wbase = wid * BPW

    pltpu.sync_copy(doc_hbm.at[pl.ds(wbase, BPW)], didx)
    pltpu.sync_copy(cidx_hbm.at[pl.ds(wbase * CTX, BPW * CTX)], cidx)
    pltpu.sync_copy(sidx_hbm.at[pl.ds(wbase * SAMP, BPW * SAMP)], sidx)

    gsems = (gsem0, gsem1)
    osems = (osem0, osem1)

    def fire(j, slot):
        base = j * NB
        hs = [pltpu.async_copy(
            para_hbm.at[didx.at[pl.ds(base, NB)]], drows.at[slot],
            gsems[slot])]
        for off, sz in ((0, 128), (128, 128), (256, 64)):
            hs.append(pltpu.async_copy(
                word_hbm.at[cidx.at[pl.ds(base * CTX + off, sz)]],
                crows.at[slot].at[pl.ds(off, sz)], gsems[slot]))
            hs.append(pltpu.async_copy(
                outt_hbm.at[sidx.at[pl.ds(base * SAMP + off, sz)]],
                srows.at[slot].at[pl.ds(off, sz)], gsems[slot]))
        return hs

    def compute(slot):
        dr = drows.at[slot]
        cr = crows.at[slot]
        sr = srows.at[slot]
        lgs = lg.at[slot]
        lane16 = lax.iota(jnp.int32, VL) * VL

        def body(b, carry):
            acc = [dr[b, pl.ds(k * VL, VL)] for k in range(4)]
            for c in range(CTX):
                r = b * CTX + c
                for k in range(4):
                    acc[k] = acc[k] + cr[r, pl.ds(k * VL, VL)]
            for s in range(SAMP):
                pbuf[pl.ds(s * VL, VL)] = _dot_partial(acc, sr, b * SAMP + s)
            for g in range(2):
                r = plsc.load_gather(pbuf, [lane16 + g * VL * VL])
                for k in range(1, VL):
                    r = r + plsc.load_gather(pbuf, [lane16 + (g * VL * VL + k)])
                lgs[b, pl.ds(g * VL, VL)] = r
            return carry

        lax.fori_loop(0, NB, body, 0)

    handles = [None, None]
    out_handles = [None, None]
    handles[0] = fire(0, 0)
    for j in range(NBLK):
        slot = j & 1
        if j + 1 < NBLK:
            handles[1 - slot] = fire(j + 1, 1 - slot)
        for h in handles[slot]:
            h.wait()
        if out_handles[slot] is not None:
            out_handles[slot].wait()
        compute(slot)
        out_handles[slot] = pltpu.async_copy(
            lg.at[slot], out_hbm.at[pl.ds(wbase + j * NB, NB)], osems[slot])
    for oh in out_handles:
        if oh is not None:
            oh.wait()


_sc_kernel = functools.partial(
    pl.kernel,
    out_type=jax.ShapeDtypeStruct((B, 2 * VL), jnp.float32),
    mesh=plsc.VectorSubcoreMesh(core_axis_name="c", subcore_axis_name="s"),
    compiler_params=pltpu.CompilerParams(
        needs_layout_passes=False, use_tc_tiling_on_sc=False),
    scratch_types=[
        pltpu.VMEM((BPW,), jnp.int32),
        pltpu.VMEM((BPW * CTX,), jnp.int32),
        pltpu.VMEM((BPW * SAMP,), jnp.int32),
        pltpu.VMEM((2, NB, D), jnp.float32),
        pltpu.VMEM((2, ROWS, D), jnp.float32),
        pltpu.VMEM((2, ROWS, D), jnp.float32),
        pltpu.VMEM((2, NB, 2 * VL), jnp.float32),
        pltpu.VMEM((2 * VL * VL,), jnp.float32),
        pltpu.SemaphoreType.DMA,
        pltpu.SemaphoreType.DMA,
        pltpu.SemaphoreType.DMA,
        pltpu.SemaphoreType.DMA,
    ],
)(_sc_body)


def kernel(doc_ids, context_ids, sample_ids, paragraph_matrix, word_matrix,
           outputs):
    doc_i = doc_ids.astype(jnp.int32)
    ctx_i = context_ids.astype(jnp.int32).reshape(-1)
    samp_i = sample_ids.astype(jnp.int32).reshape(-1)
    outt = outputs.T
    padded = _sc_kernel(doc_i, ctx_i, samp_i, paragraph_matrix,
                        word_matrix, outt)
    return padded[:, :SAMP]
